# baseline (speedup 1.0000x reference)
import functools

import jax
import jax.numpy as jnp
from jax import lax
from jax.experimental import pallas as pl
from jax.experimental.pallas import tpu as pltpu
from jax.experimental.pallas import tpu_sc as plsc

N_SPECIES = 1000000
EMBED_DIM = 32
BATCH = 16384
OUT_DIM = EMBED_DIM + 3

NC = 2
NS = 16
NW = NC * NS
BPW = BATCH // NW
IDX_CHUNK = 128
N_CHUNKS = BPW // IDX_CHUNK


def _body(idx_hbm, g_hbm, s_hbm, c_hbm, table_hbm, out_hbm,
          idx_v, emb_v, g_v, s_v, c_v, out_v, sem):
    c = lax.axis_index("c")
    s = lax.axis_index("s")
    wid = s * NC + c
    base = wid * BPW

    for j in range(N_CHUNKS):
        pltpu.sync_copy(idx_hbm.at[pl.ds(base + IDX_CHUNK * j, IDX_CHUNK)],
                        idx_v.at[j])

    cps = [
        pltpu.async_copy(table_hbm.at[idx_v.at[j]],
                         emb_v.at[pl.ds(IDX_CHUNK * j, IDX_CHUNK)], sem)
        for j in range(N_CHUNKS)
    ]

    pltpu.sync_copy(g_hbm.at[pl.ds(base, BPW)], g_v)
    pltpu.sync_copy(s_hbm.at[pl.ds(base, BPW)], s_v)
    pltpu.sync_copy(c_hbm.at[pl.ds(base, BPW)], c_v)

    for cp in cps:
        cp.wait()

    lanes = lax.iota(jnp.int32, 16)

    def row_block(it, _):
        for r in range(8):
            i = it * 8 + r
            dst = i * OUT_DIM + lanes
            plsc.store_scatter(out_v, [dst], emb_v[i, pl.ds(0, 16)])
            plsc.store_scatter(out_v, [dst + 16], emb_v[i, pl.ds(16, 16)])
        return 0

    lax.fori_loop(0, BPW // 8, row_block, 0, unroll=False)

    for gblk in range(BPW // 16):
        dst = (16 * gblk + lanes) * OUT_DIM + EMBED_DIM
        plsc.store_scatter(out_v, [dst], g_v[pl.ds(16 * gblk, 16)])
        plsc.store_scatter(out_v, [dst + 1], s_v[pl.ds(16 * gblk, 16)])
        plsc.store_scatter(out_v, [dst + 2], c_v[pl.ds(16 * gblk, 16)])

    pltpu.sync_copy(out_v, out_hbm.at[pl.ds(base * OUT_DIM, BPW * OUT_DIM)])


@functools.partial(jax.jit, static_argnums=())
def kernel(species_idx, group_idx, sin_date, cos_date, species_embedding):
    mesh = plsc.VectorSubcoreMesh(core_axis_name="c", subcore_axis_name="s")
    run = pl.kernel(
        _body,
        mesh=mesh,
        compiler_params=pltpu.CompilerParams(needs_layout_passes=False,
                                             use_tc_tiling_on_sc=False),
        out_type=jax.ShapeDtypeStruct((BATCH * OUT_DIM,), jnp.float32),
        scratch_types=[
            pltpu.VMEM((N_CHUNKS, IDX_CHUNK), jnp.int32),
            pltpu.VMEM((BPW, EMBED_DIM), jnp.float32),
            pltpu.VMEM((BPW,), jnp.float32),
            pltpu.VMEM((BPW,), jnp.float32),
            pltpu.VMEM((BPW,), jnp.float32),
            pltpu.VMEM((BPW * OUT_DIM,), jnp.float32),
            pltpu.SemaphoreType.DMA,
        ],
    )
    flat = run(species_idx.astype(jnp.int32), group_idx, sin_date, cos_date,
               species_embedding)
    return flat.reshape(BATCH, OUT_DIM)
